# baseline (speedup 1.0000x reference)
import jax
import jax.numpy as jnp
from jax import lax
from jax.experimental import pallas as pl
from jax.experimental.pallas import tpu as pltpu
from jax.experimental.pallas import tpu_sc as plsc

B = 16384
D_DRV, D_WEEK, D_TIME = 16, 3, 8
D_OUT = D_DRV + D_WEEK + D_TIME + 1
V_DRV, V_WEEK, V_TIME = 24000, 7, 1440
WK_BASE = D_DRV * V_DRV
TM_BASE = WK_BASE + 24
COMBO = TM_BASE + D_TIME * V_TIME

DIST_MEAN = 10.0
DIST_STD = 5.0


def _body(drv_idx_hbm, wk_idx_hbm, tm_idx_hbm, dist_hbm, tab_hbm, out_hbm,
          tab_v, idx_v, val_v, out_v, sem):
    wid = lax.axis_index("s") * 2 + lax.axis_index("c")

    def gather_loop(off):
        @plsc.parallel_loop(0, B, step=16, unroll=8)
        def _(i):
            idx = idx_v[pl.ds(i, 16)]
            out_v[pl.ds(i, 16)] = plsc.load_gather(tab_v, [idx + off])

    @pl.when(wid < D_DRV)
    def _():
        c1 = pltpu.async_copy(
            tab_hbm.at[pl.ds(wid * V_DRV, V_DRV)], tab_v, sem)
        c2 = pltpu.async_copy(drv_idx_hbm, idx_v, sem)
        c1.wait()
        c2.wait()
        gather_loop(jnp.full((16,), 0, jnp.int32))

    @pl.when(jnp.logical_and(wid >= D_DRV, wid < D_DRV + D_WEEK))
    def _():
        c1 = pltpu.async_copy(
            tab_hbm.at[pl.ds(WK_BASE, 24)], tab_v.at[pl.ds(0, 24)], sem)
        c2 = pltpu.async_copy(wk_idx_hbm, idx_v, sem)
        c1.wait()
        c2.wait()
        gather_loop(jnp.full((16,), (wid - D_DRV) * V_WEEK, jnp.int32))

    @pl.when(jnp.logical_and(wid >= D_DRV + D_WEEK, wid < D_OUT - 1))
    def _():
        c1 = pltpu.async_copy(
            tab_hbm.at[pl.ds(TM_BASE + (wid - (D_DRV + D_WEEK)) * V_TIME,
                             V_TIME)],
            tab_v.at[pl.ds(0, V_TIME)], sem)
        c2 = pltpu.async_copy(tm_idx_hbm, idx_v, sem)
        c1.wait()
        c2.wait()
        gather_loop(jnp.full((16,), 0, jnp.int32))

    @pl.when(wid == D_OUT - 1)
    def _():
        pltpu.async_copy(dist_hbm, val_v, sem).wait()

        @plsc.parallel_loop(0, B, step=16, unroll=8)
        def _(i):
            dv = val_v[pl.ds(i, 16)]
            out_v[pl.ds(i, 16)] = (
                dv * (1.0 / DIST_STD) - (DIST_MEAN / DIST_STD))

    @pl.when(wid < D_OUT)
    def _():
        pltpu.sync_copy(out_v, out_hbm.at[wid])


@jax.jit
def kernel(driverID, weekID, timeID, dist, driver_em, week_em, time_em):
    combo = jnp.concatenate([
        driver_em.T.reshape(-1),
        jnp.pad(week_em.T.reshape(-1), (0, 3)),
        time_em.T.reshape(-1),
    ])
    mesh = plsc.VectorSubcoreMesh(core_axis_name="c", subcore_axis_name="s")
    k = pl.kernel(
        _body,
        out_type=jax.ShapeDtypeStruct((D_OUT, B), jnp.float32),
        mesh=mesh,
        compiler_params=pltpu.CompilerParams(
            needs_layout_passes=False, use_tc_tiling_on_sc=False),
        scratch_types=[
            pltpu.VMEM((V_DRV,), jnp.float32),
            pltpu.VMEM((B,), jnp.int32),
            pltpu.VMEM((B,), jnp.float32),
            pltpu.VMEM((B,), jnp.float32),
            pltpu.SemaphoreType.DMA,
        ],
    )
    out_t = k(driverID, weekID, timeID, dist, combo)
    return out_t.T


# speedup vs baseline: 1.2140x; 1.2140x over previous
import jax
import jax.numpy as jnp
from jax import lax
from jax.experimental import pallas as pl
from jax.experimental.pallas import tpu as pltpu
from jax.experimental.pallas import tpu_sc as plsc

B = 16384
D_DRV, D_WEEK, D_TIME = 16, 3, 8
D_OUT = D_DRV + D_WEEK + D_TIME + 1
V_DRV, V_TIME = 24000, 1440
NQ = 4
QB = B // NQ

DIST_MEAN = 10.0
DIST_STD = 5.0


def _body(drv_idx_hbm, wk_idx_hbm, tm_idx_hbm, dist_hbm,
          drv_tab_hbm, wk_tab_hbm, tm_tab_hbm, out_hbm,
          tab_v, wk_tab_v, idx_v, val_v, out_v, sem, sem_w):
    wid = lax.axis_index("s") * 2 + lax.axis_index("c")

    def pipelined(idx_hbm, stage_tab, gather16):
        c_tab = stage_tab()
        idx_c = [pltpu.async_copy(idx_hbm.at[pl.ds(q * QB, QB)],
                                  idx_v.at[pl.ds(q * QB, QB)], sem)
                 for q in range(NQ)]
        c_tab.wait()
        writes = []
        for q in range(NQ):
            idx_c[q].wait()

            @plsc.parallel_loop(q * QB, (q + 1) * QB, step=16, unroll=8)
            def _(i):
                out_v[pl.ds(i, 16)] = gather16(idx_v[pl.ds(i, 16)])

            writes.append(pltpu.async_copy(
                out_v.at[pl.ds(q * QB, QB)],
                out_hbm.at[wid, pl.ds(q * QB, QB)], sem_w))
        for w in writes:
            w.wait()

    @pl.when(wid < D_DRV)
    def _():
        pipelined(
            drv_idx_hbm,
            lambda: pltpu.async_copy(drv_tab_hbm.at[wid], tab_v, sem),
            lambda idx: plsc.load_gather(tab_v, [idx]))

    @pl.when(jnp.logical_and(wid >= D_DRV, wid < D_DRV + D_WEEK))
    def _():
        wrow = jnp.full((16,), wid - D_DRV, jnp.int32)
        pipelined(
            wk_idx_hbm,
            lambda: pltpu.async_copy(wk_tab_hbm, wk_tab_v, sem),
            lambda idx: plsc.load_gather(wk_tab_v, [wrow, idx]))

    @pl.when(jnp.logical_and(wid >= D_DRV + D_WEEK, wid < D_OUT - 1))
    def _():
        pipelined(
            tm_idx_hbm,
            lambda: pltpu.async_copy(
                tm_tab_hbm.at[wid - (D_DRV + D_WEEK)],
                tab_v.at[pl.ds(0, V_TIME)], sem),
            lambda idx: plsc.load_gather(tab_v, [idx]))

    @pl.when(wid == D_OUT - 1)
    def _():
        dist_c = [pltpu.async_copy(dist_hbm.at[pl.ds(q * QB, QB)],
                                   val_v.at[pl.ds(q * QB, QB)], sem)
                  for q in range(NQ)]
        writes = []
        for q in range(NQ):
            dist_c[q].wait()

            @plsc.parallel_loop(q * QB, (q + 1) * QB, step=16, unroll=8)
            def _(i):
                dv = val_v[pl.ds(i, 16)]
                out_v[pl.ds(i, 16)] = (
                    dv * (1.0 / DIST_STD) - (DIST_MEAN / DIST_STD))

            writes.append(pltpu.async_copy(
                out_v.at[pl.ds(q * QB, QB)],
                out_hbm.at[wid, pl.ds(q * QB, QB)], sem_w))
        for w in writes:
            w.wait()


@jax.jit
def kernel(driverID, weekID, timeID, dist, driver_em, week_em, time_em):
    mesh = plsc.VectorSubcoreMesh(core_axis_name="c", subcore_axis_name="s")
    k = pl.kernel(
        _body,
        out_type=jax.ShapeDtypeStruct((D_OUT, B), jnp.float32),
        mesh=mesh,
        compiler_params=pltpu.CompilerParams(
            needs_layout_passes=False, use_tc_tiling_on_sc=False),
        scratch_types=[
            pltpu.VMEM((V_DRV,), jnp.float32),
            pltpu.VMEM((D_WEEK, 7), jnp.float32),
            pltpu.VMEM((B,), jnp.int32),
            pltpu.VMEM((B,), jnp.float32),
            pltpu.VMEM((B,), jnp.float32),
            pltpu.SemaphoreType.DMA,
            pltpu.SemaphoreType.DMA,
        ],
    )
    out_t = k(driverID, weekID, timeID, dist,
              driver_em.T, week_em.T, time_em.T)
    return out_t.T


# speedup vs baseline: 1.2575x; 1.0358x over previous
import jax
import jax.numpy as jnp
from jax import lax
from jax.experimental import pallas as pl
from jax.experimental.pallas import tpu as pltpu
from jax.experimental.pallas import tpu_sc as plsc

B = 16384
D_DRV, D_WEEK, D_TIME = 16, 3, 8
D_OUT = D_DRV + D_WEEK + D_TIME + 1
V_DRV, V_TIME = 24000, 1440
TJ_DRV = 188
TJ_TM = 12

DIST_MEAN = 10.0
DIST_STD = 5.0


def _body(drv_idx_hbm, wk_idx_hbm, tm_idx_hbm, dist_hbm,
          drv_tab_hbm, wk_tab_hbm, tm_tab_hbm, out_hbm,
          tab_v, wk_tab_v, idx_v, val_v, out_v, sem):
    wid = lax.axis_index("s") * 2 + lax.axis_index("c")

    def gather_loop(tab_ref):
        @plsc.parallel_loop(0, B, step=16, unroll=8)
        def _(i):
            idx = idx_v[pl.ds(i, 16)]
            out_v[pl.ds(i, 16)] = plsc.load_gather(
                tab_ref, [lax.shift_right_logical(idx, 7),
                          lax.bitwise_and(idx, 127)])

    @pl.when(wid < D_DRV)
    def _():
        c1 = pltpu.async_copy(
            drv_tab_hbm.at[wid // 8, :, wid % 8, :], tab_v, sem)
        c2 = pltpu.async_copy(drv_idx_hbm, idx_v, sem)
        c1.wait()
        c2.wait()
        gather_loop(tab_v)

    @pl.when(jnp.logical_and(wid >= D_DRV, wid < D_DRV + D_WEEK))
    def _():
        c1 = pltpu.async_copy(wk_tab_hbm, wk_tab_v, sem)
        c2 = pltpu.async_copy(wk_idx_hbm, idx_v, sem)
        c1.wait()
        c2.wait()
        wrow = jnp.full((16,), wid - D_DRV, jnp.int32)

        @plsc.parallel_loop(0, B, step=16, unroll=8)
        def _(i):
            idx = idx_v[pl.ds(i, 16)]
            out_v[pl.ds(i, 16)] = plsc.load_gather(wk_tab_v, [wrow, idx])

    @pl.when(jnp.logical_and(wid >= D_DRV + D_WEEK, wid < D_OUT - 1))
    def _():
        f = wid - (D_DRV + D_WEEK)
        c1 = pltpu.async_copy(
            tm_tab_hbm.at[0, :, f, :],
            tab_v.at[pl.ds(0, TJ_TM), :], sem)
        c2 = pltpu.async_copy(tm_idx_hbm, idx_v, sem)
        c1.wait()
        c2.wait()
        gather_loop(tab_v)

    @pl.when(wid == D_OUT - 1)
    def _():
        pltpu.async_copy(dist_hbm, val_v, sem).wait()

        @plsc.parallel_loop(0, B, step=16, unroll=8)
        def _(i):
            dv = val_v[pl.ds(i, 16)]
            out_v[pl.ds(i, 16)] = (
                dv * (1.0 / DIST_STD) - (DIST_MEAN / DIST_STD))

    @pl.when(wid < D_OUT)
    def _():
        pltpu.sync_copy(out_v, out_hbm.at[wid])


def _tileblock(t, tj):
    f, v = t.shape
    fp = -(-f // 8) * 8
    t = jnp.pad(t, ((0, fp - f), (0, tj * 128 - v)))
    return t.reshape(fp // 8, 8, tj, 128).transpose(0, 2, 1, 3)


@jax.jit
def kernel(driverID, weekID, timeID, dist, driver_em, week_em, time_em):
    mesh = plsc.VectorSubcoreMesh(core_axis_name="c", subcore_axis_name="s")
    k = pl.kernel(
        _body,
        out_type=jax.ShapeDtypeStruct((D_OUT, B), jnp.float32),
        mesh=mesh,
        compiler_params=pltpu.CompilerParams(
            needs_layout_passes=False, use_tc_tiling_on_sc=False),
        scratch_types=[
            pltpu.VMEM((TJ_DRV, 128), jnp.float32),
            pltpu.VMEM((D_WEEK, 7), jnp.float32),
            pltpu.VMEM((B,), jnp.int32),
            pltpu.VMEM((B,), jnp.float32),
            pltpu.VMEM((B,), jnp.float32),
            pltpu.SemaphoreType.DMA,
        ],
    )
    out_t = k(driverID, weekID, timeID, dist,
              _tileblock(driver_em.T, TJ_DRV),
              week_em.T,
              _tileblock(time_em.T, TJ_TM))
    return out_t.T


# speedup vs baseline: 1.2919x; 1.0274x over previous
import jax
import jax.numpy as jnp
from jax import lax
from jax.experimental import pallas as pl
from jax.experimental.pallas import tpu as pltpu
from jax.experimental.pallas import tpu_sc as plsc

B = 16384
D_DRV, D_WEEK, D_TIME = 16, 3, 8
D_OUT = D_DRV + D_WEEK + D_TIME + 1
V_DRV, V_TIME = 24000, 1440
TJ_DRV = 188
TJ_TM = 12

DIST_MEAN = 10.0
DIST_STD = 5.0


def _body(drv_idx_hbm, wk_idx_hbm, tm_idx_hbm, dist_hbm,
          drv_tab_hbm, tm_tab_hbm, out_hbm,
          tab_v, idx_v, val_v, out_v, sem):
    wid = lax.axis_index("s") * 2 + lax.axis_index("c")

    def gather_loop(off):
        @plsc.parallel_loop(0, B, step=16, unroll=8)
        def _(i):
            idx = idx_v[pl.ds(i, 16)] + off
            out_v[pl.ds(i, 16)] = plsc.load_gather(
                tab_v, [lax.shift_right_logical(idx, 7),
                        lax.bitwise_and(idx, 127)])

    @pl.when(wid < D_DRV)
    def _():
        c1 = pltpu.async_copy(
            drv_tab_hbm.at[wid // 8, :, wid % 8, :], tab_v, sem)
        c2 = pltpu.async_copy(drv_idx_hbm, idx_v, sem)
        c1.wait()
        c2.wait()
        gather_loop(0)

    @pl.when(jnp.logical_and(wid >= D_DRV, wid < D_DRV + D_WEEK))
    def _():
        c1 = pltpu.async_copy(
            tm_tab_hbm.at[0, :, wid - D_DRV, :],
            tab_v.at[pl.ds(0, TJ_TM), :], sem)
        c2 = pltpu.async_copy(wk_idx_hbm, idx_v, sem)
        c1.wait()
        c2.wait()
        gather_loop(V_TIME)

    @pl.when(jnp.logical_and(wid >= D_DRV + D_WEEK, wid < D_OUT - 1))
    def _():
        c1 = pltpu.async_copy(
            tm_tab_hbm.at[0, :, wid - (D_DRV + D_WEEK), :],
            tab_v.at[pl.ds(0, TJ_TM), :], sem)
        c2 = pltpu.async_copy(tm_idx_hbm, idx_v, sem)
        c1.wait()
        c2.wait()
        gather_loop(0)

    @pl.when(wid == D_OUT - 1)
    def _():
        pltpu.async_copy(dist_hbm, val_v, sem).wait()

        @plsc.parallel_loop(0, B, step=16, unroll=8)
        def _(i):
            dv = val_v[pl.ds(i, 16)]
            out_v[pl.ds(i, 16)] = (
                dv * (1.0 / DIST_STD) - (DIST_MEAN / DIST_STD))

    @pl.when(wid < D_OUT)
    def _():
        pltpu.sync_copy(out_v, out_hbm.at[wid])


def _tileblock(t, tj):
    f, v = t.shape
    fp = -(-f // 8) * 8
    t = jnp.pad(t, ((0, fp - f), (0, tj * 128 - v)))
    return t.reshape(fp // 8, 8, tj, 128).transpose(0, 2, 1, 3)


@jax.jit
def kernel(driverID, weekID, timeID, dist, driver_em, week_em, time_em):
    tm_plus = jnp.pad(time_em.T, ((0, 0), (0, 96)))
    tm_plus = lax.dynamic_update_slice(tm_plus, week_em.T, (0, V_TIME))
    mesh = plsc.VectorSubcoreMesh(core_axis_name="c", subcore_axis_name="s")
    k = pl.kernel(
        _body,
        out_type=jax.ShapeDtypeStruct((D_OUT, B), jnp.float32),
        mesh=mesh,
        compiler_params=pltpu.CompilerParams(
            needs_layout_passes=False, use_tc_tiling_on_sc=False),
        scratch_types=[
            pltpu.VMEM((TJ_DRV, 128), jnp.float32),
            pltpu.VMEM((B,), jnp.int32),
            pltpu.VMEM((B,), jnp.float32),
            pltpu.VMEM((B,), jnp.float32),
            pltpu.SemaphoreType.DMA,
        ],
    )
    out_t = k(driverID, weekID, timeID, dist,
              _tileblock(driver_em.T, TJ_DRV),
              tm_plus.reshape(1, 8, TJ_TM, 128).transpose(0, 2, 1, 3))
    return out_t.T


# speedup vs baseline: 1.3085x; 1.0129x over previous
import jax
import jax.numpy as jnp
from jax import lax
from jax.experimental import pallas as pl
from jax.experimental.pallas import tpu as pltpu
from jax.experimental.pallas import tpu_sc as plsc

B = 16384
D_DRV, D_WEEK, D_TIME = 16, 3, 8
D_OUT = D_DRV + D_WEEK + D_TIME + 1
V_DRV, V_TIME = 24000, 1440
TJ_DRV = 188
TJ_TM = 12

DIST_MEAN = 10.0
DIST_STD = 5.0


def _body(drv_idx_hbm, wk_idx_hbm, tm_idx_hbm, dist_hbm,
          drv_tab_hbm, tm_tab_hbm, out_hbm,
          tab_v, idx_v, val_v, out_v, sem):
    wid = lax.axis_index("s") * 2 + lax.axis_index("c")

    def gather_loop(off):
        @plsc.parallel_loop(0, B, step=16, unroll=8)
        def _(i):
            idx = idx_v[pl.ds(i, 16)] + off
            out_v[i // 128, pl.ds(i % 128, 16)] = plsc.load_gather(
                tab_v, [lax.shift_right_logical(idx, 7),
                        lax.bitwise_and(idx, 127)])

    @pl.when(wid < D_DRV)
    def _():
        c1 = pltpu.async_copy(
            drv_tab_hbm.at[wid // 8, :, wid % 8, :], tab_v, sem)
        c2 = pltpu.async_copy(drv_idx_hbm, idx_v, sem)
        c1.wait()
        c2.wait()
        gather_loop(0)

    @pl.when(jnp.logical_and(wid >= D_DRV, wid < D_DRV + D_WEEK))
    def _():
        c1 = pltpu.async_copy(
            tm_tab_hbm.at[0, :, wid - D_DRV, :],
            tab_v.at[pl.ds(0, TJ_TM), :], sem)
        c2 = pltpu.async_copy(wk_idx_hbm, idx_v, sem)
        c1.wait()
        c2.wait()
        gather_loop(V_TIME)

    @pl.when(jnp.logical_and(wid >= D_DRV + D_WEEK, wid < D_OUT - 1))
    def _():
        c1 = pltpu.async_copy(
            tm_tab_hbm.at[0, :, wid - (D_DRV + D_WEEK), :],
            tab_v.at[pl.ds(0, TJ_TM), :], sem)
        c2 = pltpu.async_copy(tm_idx_hbm, idx_v, sem)
        c1.wait()
        c2.wait()
        gather_loop(0)

    @pl.when(wid == D_OUT - 1)
    def _():
        pltpu.async_copy(dist_hbm, val_v, sem).wait()

        @plsc.parallel_loop(0, B, step=16, unroll=8)
        def _(i):
            dv = val_v[pl.ds(i, 16)]
            out_v[i // 128, pl.ds(i % 128, 16)] = (
                dv * (1.0 / DIST_STD) - (DIST_MEAN / DIST_STD))

    @pl.when(wid < D_OUT)
    def _():
        pltpu.sync_copy(out_v, out_hbm.at[wid // 8, :, wid % 8, :])


def _tileblock(t, tj):
    f, v = t.shape
    fp = -(-f // 8) * 8
    t = jnp.pad(t, ((0, fp - f), (0, tj * 128 - v)))
    return t.reshape(fp // 8, 8, tj, 128).transpose(0, 2, 1, 3)


@jax.jit
def kernel(driverID, weekID, timeID, dist, driver_em, week_em, time_em):
    tm_plus = jnp.pad(time_em.T, ((0, 0), (0, 96)))
    tm_plus = lax.dynamic_update_slice(tm_plus, week_em.T, (0, V_TIME))
    mesh = plsc.VectorSubcoreMesh(core_axis_name="c", subcore_axis_name="s")
    k = pl.kernel(
        _body,
        out_type=jax.ShapeDtypeStruct((4, 128, 8, 128), jnp.float32),
        mesh=mesh,
        compiler_params=pltpu.CompilerParams(
            needs_layout_passes=False, use_tc_tiling_on_sc=False),
        scratch_types=[
            pltpu.VMEM((TJ_DRV, 128), jnp.float32),
            pltpu.VMEM((B,), jnp.int32),
            pltpu.VMEM((B,), jnp.float32),
            pltpu.VMEM((128, 128), jnp.float32),
            pltpu.SemaphoreType.DMA,
        ],
    )
    out4 = k(driverID, weekID, timeID, dist,
             _tileblock(driver_em.T, TJ_DRV),
             tm_plus.reshape(1, 8, TJ_TM, 128).transpose(0, 2, 1, 3))
    out_t = out4.transpose(0, 2, 1, 3).reshape(32, B)
    return out_t[:D_OUT].T
